# baseline (device time: 4012218 ns/iter reference)
import jax
import jax.numpy as jnp
from jax import lax
from jax.experimental import pallas as pl
from jax.experimental.pallas import tpu as pltpu

N_DEV = 16
N_ROUNDS = 8


def kernel(x):
    m_per, n = x.shape
    x16 = x.astype(jnp.bfloat16)

    def body(
        x_ref, out_ref, comm_ref, copy_sem, send_r, recv_r, send_l, recv_l,
        copy_r, copy_l,
    ):
        my = lax.axis_index("i")
        left = lax.rem(my - 1 + N_DEV, N_DEV)
        right = lax.rem(my + 1, N_DEV)

        barrier_sem = pltpu.get_barrier_semaphore()
        for nbr in (left, right):
            pl.semaphore_signal(
                barrier_sem, inc=1,
                device_id=(nbr,), device_id_type=pl.DeviceIdType.MESH,
            )
        pl.semaphore_wait(barrier_sem, 2)

        local_copy = pltpu.make_async_copy(
            x_ref, out_ref.at[pl.ds(my * m_per, m_per), :], copy_sem
        )
        local_copy.start()

        half = m_per // 2
        pending = []
        for h in range(N_ROUNDS):
            c_r = lax.rem(my - h + N_DEV, N_DEV)
            rows_r = m_per if h < N_ROUNDS - 1 else half
            sl_r = pl.ds(c_r * m_per, rows_r)
            src_r = x_ref if h == 0 else comm_ref.at[sl_r, :]
            rdma_r = pltpu.make_async_remote_copy(
                src_ref=src_r,
                dst_ref=comm_ref.at[sl_r, :],
                send_sem=send_r.at[h],
                recv_sem=recv_r.at[h],
                device_id=(right,),
                device_id_type=pl.DeviceIdType.MESH,
            )
            rdma_r.start()

            c_l = lax.rem(my + h, N_DEV)
            off_l = c_l * m_per if h < N_ROUNDS - 1 else c_l * m_per + half
            rows_l = m_per if h < N_ROUNDS - 1 else half
            sl_l = pl.ds(off_l, rows_l)
            src_l = x_ref if h == 0 else comm_ref.at[sl_l, :]
            rdma_l = pltpu.make_async_remote_copy(
                src_ref=src_l,
                dst_ref=comm_ref.at[sl_l, :],
                send_sem=send_l.at[h],
                recv_sem=recv_l.at[h],
                device_id=(left,),
                device_id_type=pl.DeviceIdType.MESH,
            )
            rdma_l.start()
            rdma_l.wait()
            rdma_r.wait()

            in_r = pl.ds(
                lax.rem(my - h - 1 + N_DEV, N_DEV) * m_per, rows_r
            )
            cp_r = pltpu.make_async_copy(
                comm_ref.at[in_r, :], out_ref.at[in_r, :], copy_r.at[h]
            )
            cp_r.start()
            c_in_l = lax.rem(my + h + 1, N_DEV)
            in_l = pl.ds(
                c_in_l * m_per if h < N_ROUNDS - 1 else c_in_l * m_per + half,
                rows_l,
            )
            cp_l = pltpu.make_async_copy(
                comm_ref.at[in_l, :], out_ref.at[in_l, :], copy_l.at[h]
            )
            cp_l.start()
            pending.append(cp_r)
            pending.append(cp_l)

        for cp in pending:
            cp.wait()
        local_copy.wait()

    out, _comm = pl.pallas_call(
        body,
        out_shape=[
            jax.ShapeDtypeStruct((N_DEV * m_per, n), jnp.bfloat16),
            jax.ShapeDtypeStruct((N_DEV * m_per, n), jnp.bfloat16),
        ],
        in_specs=[pl.BlockSpec(memory_space=pltpu.VMEM)],
        out_specs=[
            pl.BlockSpec(memory_space=pl.ANY),
            pl.BlockSpec(memory_space=pl.ANY),
        ],
        scratch_shapes=[
            pltpu.SemaphoreType.DMA,
            pltpu.SemaphoreType.DMA((N_ROUNDS,)),
            pltpu.SemaphoreType.DMA((N_ROUNDS,)),
            pltpu.SemaphoreType.DMA((N_ROUNDS,)),
            pltpu.SemaphoreType.DMA((N_ROUNDS,)),
            pltpu.SemaphoreType.DMA((N_ROUNDS,)),
            pltpu.SemaphoreType.DMA((N_ROUNDS,)),
        ],
        compiler_params=pltpu.CompilerParams(collective_id=0),
    )(x16)
    return out


# device time: 799864 ns/iter; 5.0161x vs baseline; 5.0161x over previous
import jax
import jax.numpy as jnp
from jax import lax
from jax.experimental import pallas as pl
from jax.experimental.pallas import tpu as pltpu

N_DEV = 16
N_ROUNDS = 8


def kernel(x):
    m_per, n = x.shape
    half = m_per // 2
    x16 = x.astype(jnp.bfloat16)

    def body(
        x_ref, out_ref, comm_r, comm_l, copy_sem,
        send_r, recv_r, send_l, recv_l, copy_r, copy_l,
    ):
        my = lax.axis_index("i")
        left = lax.rem(my - 1 + N_DEV, N_DEV)
        right = lax.rem(my + 1, N_DEV)

        barrier_sem = pltpu.get_barrier_semaphore()
        for nbr in (left, right):
            pl.semaphore_signal(
                barrier_sem, inc=1,
                device_id=(nbr,), device_id_type=pl.DeviceIdType.MESH,
            )
        pl.semaphore_wait(barrier_sem, 2)

        local_copy = pltpu.make_async_copy(
            x_ref, out_ref.at[pl.ds(my * m_per, m_per), :], copy_sem
        )
        local_copy.start()

        pending = []
        for h in range(N_ROUNDS):
            last = h == N_ROUNDS - 1
            rows = half if last else m_per
            src_r = x_ref if h == 0 else comm_r.at[(h - 1) % 2]
            rdma_r = pltpu.make_async_remote_copy(
                src_ref=src_r.at[pl.ds(0, rows), :],
                dst_ref=comm_r.at[h % 2, pl.ds(0, rows), :],
                send_sem=send_r.at[h],
                recv_sem=recv_r.at[h],
                device_id=(right,),
                device_id_type=pl.DeviceIdType.MESH,
            )
            rdma_r.start()

            off = half if last else 0
            src_l = x_ref if h == 0 else comm_l.at[(h - 1) % 2]
            rdma_l = pltpu.make_async_remote_copy(
                src_ref=src_l.at[pl.ds(off, rows), :],
                dst_ref=comm_l.at[h % 2, pl.ds(off, rows), :],
                send_sem=send_l.at[h],
                recv_sem=recv_l.at[h],
                device_id=(left,),
                device_id_type=pl.DeviceIdType.MESH,
            )
            rdma_l.start()
            rdma_l.wait()
            rdma_r.wait()

            c_r = lax.rem(my - h - 1 + N_DEV, N_DEV)
            cp_r = pltpu.make_async_copy(
                comm_r.at[h % 2, pl.ds(0, rows), :],
                out_ref.at[pl.ds(c_r * m_per, rows), :],
                copy_r.at[h],
            )
            cp_r.start()
            c_l = lax.rem(my + h + 1, N_DEV)
            cp_l = pltpu.make_async_copy(
                comm_l.at[h % 2, pl.ds(off, rows), :],
                out_ref.at[pl.ds(c_l * m_per + off, rows), :],
                copy_l.at[h],
            )
            cp_l.start()
            pending += [cp_r, cp_l]

        for cp in pending:
            cp.wait()
        local_copy.wait()

    return pl.pallas_call(
        body,
        out_shape=jax.ShapeDtypeStruct((N_DEV * m_per, n), jnp.bfloat16),
        in_specs=[pl.BlockSpec(memory_space=pltpu.VMEM)],
        out_specs=pl.BlockSpec(memory_space=pl.ANY),
        scratch_shapes=[
            pltpu.VMEM((2, m_per, n), jnp.bfloat16),
            pltpu.VMEM((2, m_per, n), jnp.bfloat16),
            pltpu.SemaphoreType.DMA,
            pltpu.SemaphoreType.DMA((N_ROUNDS,)),
            pltpu.SemaphoreType.DMA((N_ROUNDS,)),
            pltpu.SemaphoreType.DMA((N_ROUNDS,)),
            pltpu.SemaphoreType.DMA((N_ROUNDS,)),
            pltpu.SemaphoreType.DMA((N_ROUNDS,)),
            pltpu.SemaphoreType.DMA((N_ROUNDS,)),
        ],
        compiler_params=pltpu.CompilerParams(collective_id=0),
    )(x16)


# device time: 798114 ns/iter; 5.0271x vs baseline; 1.0022x over previous
import jax
import jax.numpy as jnp
from jax import lax
from jax.experimental import pallas as pl
from jax.experimental.pallas import tpu as pltpu

N_DEV = 16
N_ROUNDS = 8


def kernel(x):
    m_per, n = x.shape
    half = m_per // 2
    x16 = x.astype(jnp.bfloat16)

    def body(
        x_ref, out_ref, dummy_ref, comm_r, comm_l, copy_sem,
        send_r, recv_r, send_l, recv_l, copy_r, copy_l,
    ):
        my = lax.axis_index("i")
        left = lax.rem(my - 1 + N_DEV, N_DEV)
        right = lax.rem(my + 1, N_DEV)

        barrier_sem = pltpu.get_barrier_semaphore()
        for nbr in (left, right):
            pl.semaphore_signal(
                barrier_sem, inc=1,
                device_id=(nbr,), device_id_type=pl.DeviceIdType.MESH,
            )
        pl.semaphore_wait(barrier_sem, 2)

        local_copy = pltpu.make_async_copy(
            x_ref, out_ref.at[pl.ds(my * m_per, m_per), :], copy_sem
        )
        local_copy.start()

        pending = []
        for h in range(N_ROUNDS):
            last = h == N_ROUNDS - 1
            rows = half if last else m_per
            src_r = x_ref if h == 0 else comm_r.at[(h - 1) % 2]
            rdma_r = pltpu.make_async_remote_copy(
                src_ref=src_r.at[pl.ds(0, rows), :],
                dst_ref=comm_r.at[h % 2, pl.ds(0, rows), :],
                send_sem=send_r.at[h],
                recv_sem=recv_r.at[h],
                device_id=(right,),
                device_id_type=pl.DeviceIdType.MESH,
            )
            rdma_r.start()

            off = half if last else 0
            src_l = x_ref if h == 0 else comm_l.at[(h - 1) % 2]
            rdma_l = pltpu.make_async_remote_copy(
                src_ref=src_l.at[pl.ds(off, rows), :],
                dst_ref=comm_l.at[h % 2, pl.ds(off, rows), :],
                send_sem=send_l.at[h],
                recv_sem=recv_l.at[h],
                device_id=(left,),
                device_id_type=pl.DeviceIdType.MESH,
            )
            rdma_l.start()
            rdma_l.wait()
            rdma_r.wait()

            c_r = lax.rem(my - h - 1 + N_DEV, N_DEV)
            cp_r = pltpu.make_async_copy(
                comm_r.at[h % 2, pl.ds(0, rows), :],
                out_ref.at[pl.ds(c_r * m_per, rows), :],
                copy_r.at[h],
            )
            cp_r.start()
            c_l = lax.rem(my + h + 1, N_DEV)
            cp_l = pltpu.make_async_copy(
                comm_l.at[h % 2, pl.ds(off, rows), :],
                out_ref.at[pl.ds(c_l * m_per + off, rows), :],
                copy_l.at[h],
            )
            cp_l.start()
            pending += [cp_r, cp_l]

        for cp in pending:
            cp.wait()
        local_copy.wait()

    out, _ = pl.pallas_call(
        body,
        out_shape=[
            jax.ShapeDtypeStruct((N_DEV * m_per, n), jnp.bfloat16),
            jax.ShapeDtypeStruct((8, 128), jnp.bfloat16),
        ],
        in_specs=[pl.BlockSpec(memory_space=pltpu.VMEM)],
        out_specs=[
            pl.BlockSpec(memory_space=pl.ANY),
            pl.BlockSpec(memory_space=pl.ANY),
        ],
        scratch_shapes=[
            pltpu.VMEM((2, m_per, n), jnp.bfloat16),
            pltpu.VMEM((2, m_per, n), jnp.bfloat16),
            pltpu.SemaphoreType.DMA,
            pltpu.SemaphoreType.DMA((N_ROUNDS,)),
            pltpu.SemaphoreType.DMA((N_ROUNDS,)),
            pltpu.SemaphoreType.DMA((N_ROUNDS,)),
            pltpu.SemaphoreType.DMA((N_ROUNDS,)),
            pltpu.SemaphoreType.DMA((N_ROUNDS,)),
            pltpu.SemaphoreType.DMA((N_ROUNDS,)),
        ],
        compiler_params=pltpu.CompilerParams(collective_id=0),
    )(x16)
    return out
